# baseline (speedup 1.0000x reference)
import functools

import jax
import jax.numpy as jnp
from jax import lax
from jax.experimental import pallas as pl
from jax.experimental.pallas import tpu as pltpu
from jax.experimental.pallas import tpu_sc as plsc

N = 10000
E = 160000
D = 256
DH = 128
NS = 16
EPT = E // NS
CH = 80
NCHUNK = EPT // CH
RPT = 624
RPT_LAST = N - 15 * RPT


def _sc_segment_sum(nf2, ef2, pk0, pk1, zrows):
    mesh = plsc.VectorSubcoreMesh(core_axis_name="c", subcore_axis_name="s")

    @functools.partial(
        pl.kernel,
        out_type=(
            jax.ShapeDtypeStruct((N, DH), jnp.float32),
            jax.ShapeDtypeStruct((N, DH), jnp.float32),
        ),
        mesh=mesh,
        scratch_types=[
            pltpu.VMEM_SHARED((N, DH), jnp.float32),
            pltpu.VMEM((9, CH), jnp.int32),
            pltpu.VMEM((CH, DH), jnp.float32),
            pltpu.VMEM((CH, DH), jnp.float32),
            pltpu.VMEM((CH, DH), jnp.float32),
            pltpu.VMEM((CH, DH), jnp.float32),
            pltpu.SemaphoreType.DMA,
            pltpu.SemaphoreType.DMA,
            pltpu.SemaphoreType.DMA,
            pltpu.SemaphoreType.DMA,
            pltpu.SemaphoreType.DMA,
            pltpu.SemaphoreType.DMA,
        ],
    )
    def k(nf2_h, ef2_h, pk0_h, pk1_h, zrows_h,
          out0_h, out1_h, acc, pkb,
          nr0, er0, nr1, er1,
          isem0, isem1, isem2, gsem0, gsem1, ssem):
        c = lax.axis_index("c")
        s = lax.axis_index("s")
        nrbuf = (nr0, nr1)
        erbuf = (er0, er1)
        isem = (isem0, isem1, isem2)
        gsem = (gsem0, gsem1)

        def run(pk_h, out_h):
            def idx_stage(kk, bi):
                pltpu.async_copy(pk_h.at[s * NCHUNK + kk],
                                 pkb.at[pl.ds(3 * bi, 3)], isem[bi])

            def idx_wait(bi):
                pltpu.make_async_copy(pk_h.at[0],
                                      pkb.at[pl.ds(3 * bi, 3)],
                                      isem[bi]).wait()

            def gathers(bd, bi):
                pltpu.async_copy(nf2_h.at[pkb.at[3 * bi]], nrbuf[bd],
                                 gsem[bd])
                pltpu.async_copy(ef2_h.at[pkb.at[3 * bi + 1]], erbuf[bd],
                                 gsem[bd])

            def gathers_wait(bd):
                pltpu.make_async_copy(nf2_h.at[pl.ds(0, CH)], nrbuf[bd],
                                      gsem[bd]).wait()
                pltpu.make_async_copy(ef2_h.at[pl.ds(0, CH)], erbuf[bd],
                                      gsem[bd]).wait()

            def scatter(bd, bi):
                a1 = pltpu.async_copy(nrbuf[bd], acc.at[pkb.at[3 * bi + 2]],
                                      ssem, add=True)
                a2 = pltpu.async_copy(erbuf[bd], acc.at[pkb.at[3 * bi + 2]],
                                      ssem, add=True)
                a1.wait()
                a2.wait()

            def sub(kk, bd, bi, do_next_g, do_next_i):
                if do_next_g:
                    idx_wait((bi + 1) % 3)
                    gathers((bd + 1) % 2, (bi + 1) % 3)
                if do_next_i:
                    idx_stage(kk + 2, (bi + 2) % 3)
                gathers_wait(bd)
                scatter(bd, bi)

            idx_stage(0, 0)
            idx_wait(0)
            gathers(0, 0)
            idx_stage(1, 1)

            @pl.when(s < 15)
            def _():
                pltpu.sync_copy(zrows_h.at[pl.ds(0, RPT)],
                                acc.at[pl.ds(s * RPT, RPT)])

            @pl.when(s == 15)
            def _():
                pltpu.sync_copy(zrows_h, acc.at[pl.ds(15 * RPT, RPT_LAST)])

            plsc.subcore_barrier()

            def six(i, carry):
                kk = 6 * i
                for j in range(6):
                    sub(kk + j, j % 2, j % 3, True, True)
                return carry

            nt = (NCHUNK - 4) // 6
            lax.fori_loop(0, nt, six, 0)
            for kk in range(6 * nt, NCHUNK):
                sub(kk, kk % 2, kk % 3, kk + 1 < NCHUNK, kk + 2 < NCHUNK)

            plsc.subcore_barrier()

            @pl.when(s < 15)
            def _():
                pltpu.sync_copy(acc.at[pl.ds(s * RPT, RPT)],
                                out_h.at[pl.ds(s * RPT, RPT)])

            @pl.when(s == 15)
            def _():
                pltpu.sync_copy(acc.at[pl.ds(15 * RPT, RPT_LAST)],
                                out_h.at[pl.ds(15 * RPT, RPT_LAST)])

        @pl.when(c == 0)
        def _():
            run(pk0_h, out0_h)

        @pl.when(c == 1)
        def _():
            run(pk1_h, out1_h)

    return k(nf2, ef2, pk0, pk1, zrows)


RB = 2000


def _mlp_bn_kernel(a0_ref, a1_ref, w1_ref, b1_ref, w2_ref, b2_ref,
                   g_ref, bt_ref, o_ref, h_scr, st_scr):
    p = pl.program_id(0)
    i = pl.program_id(1)

    @pl.when(p == 0)
    def _():
        x = jnp.concatenate([a0_ref[...], a1_ref[...]],
                            axis=1).astype(jnp.bfloat16)
        h1 = jnp.maximum(
            jnp.dot(x, w1_ref[...], preferred_element_type=jnp.float32)
            + b1_ref[...], 0.0)
        h = (jnp.dot(h1.astype(jnp.bfloat16), w2_ref[...],
                     preferred_element_type=jnp.float32) + b2_ref[...])
        h_scr[pl.ds(i * RB, RB), :] = h
        s1 = jnp.sum(h, axis=0, keepdims=True)
        s2 = jnp.sum(h * h, axis=0, keepdims=True)

        @pl.when(i == 0)
        def _():
            st_scr[...] = jnp.zeros_like(st_scr)

        st_scr[0:2] += jnp.concatenate([s1, s2], axis=0)

    @pl.when(p == 1)
    def _():
        inv_n = 1.0 / N
        mean = st_scr[0:1] * inv_n
        var = st_scr[1:2] * inv_n - mean * mean
        rstd = lax.rsqrt(var + 1e-5)
        hb = h_scr[pl.ds(i * RB, RB), :]
        o_ref[...] = (hb - mean) * (rstd * g_ref[...]) + bt_ref[...]


def _mlp_bn(a0, a1, W1, b1, W2, b2, gamma, beta):
    grid = (2, N // RB)
    return pl.pallas_call(
        _mlp_bn_kernel,
        grid=grid,
        in_specs=[
            pl.BlockSpec((RB, DH), lambda p, i: (i, 0)),
            pl.BlockSpec((RB, DH), lambda p, i: (i, 0)),
            pl.BlockSpec((D, 2 * D), lambda p, i: (0, 0)),
            pl.BlockSpec((1, 2 * D), lambda p, i: (0, 0)),
            pl.BlockSpec((2 * D, D), lambda p, i: (0, 0)),
            pl.BlockSpec((1, D), lambda p, i: (0, 0)),
            pl.BlockSpec((1, D), lambda p, i: (0, 0)),
            pl.BlockSpec((1, D), lambda p, i: (0, 0)),
        ],
        out_specs=pl.BlockSpec((RB, D), lambda p, i: (i, 0)),
        out_shape=jax.ShapeDtypeStruct((N, D), jnp.float32),
        scratch_shapes=[
            pltpu.VMEM((N, D), jnp.float32),
            pltpu.VMEM((8, D), jnp.float32),
        ],
    )(a0, a1, W1.astype(jnp.bfloat16), b1.reshape(1, 2 * D),
      W2.astype(jnp.bfloat16), b2.reshape(1, D),
      gamma.reshape(1, D), beta.reshape(1, D))


def kernel(node_feats, edge_index, edge_feats, W1, b1, W2, b2, gamma, beta):
    src = edge_index[0]
    dst = edge_index[1]
    nf2 = node_feats.reshape(2 * N, DH)
    ef2 = edge_feats.reshape(2 * E, DH)
    sidx0 = src * 2
    sidx1 = sidx0 + 1
    eidx0 = jnp.arange(E, dtype=jnp.int32) * 2
    eidx1 = eidx0 + 1
    pk0 = jnp.stack([sidx0.reshape(-1, CH), eidx0.reshape(-1, CH),
                     dst.reshape(-1, CH)], axis=1)
    pk1 = jnp.stack([sidx1.reshape(-1, CH), eidx1.reshape(-1, CH),
                     dst.reshape(-1, CH)], axis=1)
    zrows = jnp.zeros((RPT_LAST, DH), jnp.float32)
    a0, a1 = _sc_segment_sum(nf2, ef2, pk0, pk1, zrows)
    return _mlp_bn(a0, a1, W1, b1, W2, b2, gamma, beta)


# speedup vs baseline: 1.0085x; 1.0085x over previous
import functools

import jax
import jax.numpy as jnp
from jax import lax
from jax.experimental import pallas as pl
from jax.experimental.pallas import tpu as pltpu
from jax.experimental.pallas import tpu_sc as plsc

N = 10000
E = 160000
D = 256
DH = 128
NS = 16
EPT = E // NS
CH = 80
NCHUNK = EPT // CH
RPT = 624
RPT_LAST = N - 15 * RPT


def _sc_segment_sum(nf2, ef2, sidx0, sidx1, eidx0, eidx1, dst, zrows):
    mesh = plsc.VectorSubcoreMesh(core_axis_name="c", subcore_axis_name="s")

    @functools.partial(
        pl.kernel,
        out_type=(
            jax.ShapeDtypeStruct((N, DH), jnp.float32),
            jax.ShapeDtypeStruct((N, DH), jnp.float32),
        ),
        mesh=mesh,
        scratch_types=[
            pltpu.VMEM_SHARED((N, DH), jnp.float32),
            pltpu.VMEM((3, CH), jnp.int32),
            pltpu.VMEM((3, CH), jnp.int32),
            pltpu.VMEM((3, CH), jnp.int32),
            pltpu.VMEM((CH, DH), jnp.float32),
            pltpu.VMEM((CH, DH), jnp.float32),
            pltpu.VMEM((CH, DH), jnp.float32),
            pltpu.VMEM((CH, DH), jnp.float32),
            pltpu.SemaphoreType.DMA,
            pltpu.SemaphoreType.DMA,
            pltpu.SemaphoreType.DMA,
            pltpu.SemaphoreType.DMA,
            pltpu.SemaphoreType.DMA,
            pltpu.SemaphoreType.DMA,
        ],
    )
    def k(nf2_h, ef2_h, sidx0_h, sidx1_h, eidx0_h, eidx1_h, dst_h, zrows_h,
          out0_h, out1_h, acc, sib, eib, dib,
          nr0, er0, nr1, er1,
          isem0, isem1, isem2, gsem0, gsem1, ssem):
        c = lax.axis_index("c")
        s = lax.axis_index("s")
        nrbuf = (nr0, nr1)
        erbuf = (er0, er1)
        isem = (isem0, isem1, isem2)
        gsem = (gsem0, gsem1)

        def run(sidx_h, eidx_h, out_h):
            def idx_stage(kk, bi):
                base = s * EPT + kk * CH
                pltpu.async_copy(sidx_h.at[pl.ds(base, CH)], sib.at[bi],
                                 isem[bi])
                pltpu.async_copy(eidx_h.at[pl.ds(base, CH)], eib.at[bi],
                                 isem[bi])
                pltpu.async_copy(dst_h.at[pl.ds(base, CH)], dib.at[bi],
                                 isem[bi])

            def idx_wait(bi):
                pltpu.make_async_copy(sidx_h.at[pl.ds(0, CH)], sib.at[bi],
                                      isem[bi]).wait()
                pltpu.make_async_copy(eidx_h.at[pl.ds(0, CH)], eib.at[bi],
                                      isem[bi]).wait()
                pltpu.make_async_copy(dst_h.at[pl.ds(0, CH)], dib.at[bi],
                                      isem[bi]).wait()

            def gathers(bd, bi):
                pltpu.async_copy(nf2_h.at[sib.at[bi]], nrbuf[bd], gsem[bd])
                pltpu.async_copy(ef2_h.at[eib.at[bi]], erbuf[bd], gsem[bd])

            def gathers_wait(bd):
                pltpu.make_async_copy(nf2_h.at[pl.ds(0, CH)], nrbuf[bd],
                                      gsem[bd]).wait()
                pltpu.make_async_copy(ef2_h.at[pl.ds(0, CH)], erbuf[bd],
                                      gsem[bd]).wait()

            def combine(bd):
                def row2(r, carry):
                    for rr in range(2):
                        for cc in range(DH // 16):
                            sl = pl.ds(16 * cc, 16)
                            nrbuf[bd][2 * r + rr, sl] = (
                                nrbuf[bd][2 * r + rr, sl]
                                + erbuf[bd][2 * r + rr, sl])
                    return carry

                lax.fori_loop(0, CH // 2, row2, 0)

            def scatter(bd, bi):
                pltpu.async_copy(nrbuf[bd], acc.at[dib.at[bi]], ssem,
                                 add=True).wait()

            def sub(kk, bd, bi, do_next_g, do_next_i):
                if do_next_g:
                    idx_wait((bi + 1) % 3)
                    gathers((bd + 1) % 2, (bi + 1) % 3)
                if do_next_i:
                    idx_stage(kk + 2, (bi + 2) % 3)
                gathers_wait(bd)
                combine(bd)
                scatter(bd, bi)

            idx_stage(0, 0)
            idx_wait(0)
            gathers(0, 0)
            idx_stage(1, 1)

            @pl.when(s < 15)
            def _():
                pltpu.sync_copy(zrows_h.at[pl.ds(0, RPT)],
                                acc.at[pl.ds(s * RPT, RPT)])

            @pl.when(s == 15)
            def _():
                pltpu.sync_copy(zrows_h, acc.at[pl.ds(15 * RPT, RPT_LAST)])

            plsc.subcore_barrier()

            def six(i, carry):
                kk = 6 * i
                for j in range(6):
                    sub(kk + j, j % 2, j % 3, True, True)
                return carry

            nt = (NCHUNK - 4) // 6
            lax.fori_loop(0, nt, six, 0)
            for kk in range(6 * nt, NCHUNK):
                sub(kk, kk % 2, kk % 3, kk + 1 < NCHUNK, kk + 2 < NCHUNK)

            plsc.subcore_barrier()

            @pl.when(s < 15)
            def _():
                pltpu.sync_copy(acc.at[pl.ds(s * RPT, RPT)],
                                out_h.at[pl.ds(s * RPT, RPT)])

            @pl.when(s == 15)
            def _():
                pltpu.sync_copy(acc.at[pl.ds(15 * RPT, RPT_LAST)],
                                out_h.at[pl.ds(15 * RPT, RPT_LAST)])

        @pl.when(c == 0)
        def _():
            run(sidx0_h, eidx0_h, out0_h)

        @pl.when(c == 1)
        def _():
            run(sidx1_h, eidx1_h, out1_h)

    return k(nf2, ef2, sidx0, sidx1, eidx0, eidx1, dst, zrows)


RB = 2000


def _mlp_bn_kernel(a0_ref, a1_ref, w1_ref, b1_ref, w2_ref, b2_ref,
                   g_ref, bt_ref, o_ref, h_scr, st_scr):
    p = pl.program_id(0)
    i = pl.program_id(1)

    @pl.when(p == 0)
    def _():
        x = jnp.concatenate([a0_ref[...], a1_ref[...]],
                            axis=1).astype(jnp.bfloat16)
        h1 = jnp.maximum(
            jnp.dot(x, w1_ref[...], preferred_element_type=jnp.float32)
            + b1_ref[...], 0.0)
        h = (jnp.dot(h1.astype(jnp.bfloat16), w2_ref[...],
                     preferred_element_type=jnp.float32) + b2_ref[...])
        h_scr[pl.ds(i * RB, RB), :] = h
        s1 = jnp.sum(h, axis=0, keepdims=True)
        s2 = jnp.sum(h * h, axis=0, keepdims=True)

        @pl.when(i == 0)
        def _():
            st_scr[...] = jnp.zeros_like(st_scr)

        st_scr[0:2] += jnp.concatenate([s1, s2], axis=0)

    @pl.when(p == 1)
    def _():
        inv_n = 1.0 / N
        mean = st_scr[0:1] * inv_n
        var = st_scr[1:2] * inv_n - mean * mean
        rstd = lax.rsqrt(var + 1e-5)
        hb = h_scr[pl.ds(i * RB, RB), :]
        o_ref[...] = (hb - mean) * (rstd * g_ref[...]) + bt_ref[...]


def _mlp_bn(a0, a1, W1, b1, W2, b2, gamma, beta):
    grid = (2, N // RB)
    return pl.pallas_call(
        _mlp_bn_kernel,
        grid=grid,
        in_specs=[
            pl.BlockSpec((RB, DH), lambda p, i: (i, 0)),
            pl.BlockSpec((RB, DH), lambda p, i: (i, 0)),
            pl.BlockSpec((D, 2 * D), lambda p, i: (0, 0)),
            pl.BlockSpec((1, 2 * D), lambda p, i: (0, 0)),
            pl.BlockSpec((2 * D, D), lambda p, i: (0, 0)),
            pl.BlockSpec((1, D), lambda p, i: (0, 0)),
            pl.BlockSpec((1, D), lambda p, i: (0, 0)),
            pl.BlockSpec((1, D), lambda p, i: (0, 0)),
        ],
        out_specs=pl.BlockSpec((RB, D), lambda p, i: (i, 0)),
        out_shape=jax.ShapeDtypeStruct((N, D), jnp.float32),
        scratch_shapes=[
            pltpu.VMEM((N, D), jnp.float32),
            pltpu.VMEM((8, D), jnp.float32),
        ],
    )(a0, a1, W1.astype(jnp.bfloat16), b1.reshape(1, 2 * D),
      W2.astype(jnp.bfloat16), b2.reshape(1, D),
      gamma.reshape(1, D), beta.reshape(1, D))


def kernel(node_feats, edge_index, edge_feats, W1, b1, W2, b2, gamma, beta):
    src = edge_index[0]
    dst = edge_index[1]
    nf2 = node_feats.reshape(2 * N, DH)
    ef2 = edge_feats.reshape(2 * E, DH)
    sidx0 = src * 2
    sidx1 = sidx0 + 1
    eidx0 = jnp.arange(E, dtype=jnp.int32) * 2
    eidx1 = eidx0 + 1
    zrows = jnp.zeros((RPT_LAST, DH), jnp.float32)
    a0, a1 = _sc_segment_sum(nf2, ef2, sidx0, sidx1, eidx0, eidx1, dst, zrows)
    return _mlp_bn(a0, a1, W1, b1, W2, b2, gamma, beta)


# speedup vs baseline: 1.0237x; 1.0150x over previous
import functools

import jax
import jax.numpy as jnp
from jax import lax
from jax.experimental import pallas as pl
from jax.experimental.pallas import tpu as pltpu
from jax.experimental.pallas import tpu_sc as plsc

N = 10000
E = 160000
D = 256
DH = 128
NS = 16
EPT = E // NS
CH = 80
NCHUNK = EPT // CH
RPT = 624
RPT_LAST = N - 15 * RPT


def _sc_segment_sum(nf2, ef2, sidx0, sidx1, eidx0, eidx1, dst, zrows):
    mesh = plsc.VectorSubcoreMesh(core_axis_name="c", subcore_axis_name="s")

    @functools.partial(
        pl.kernel,
        out_type=(
            jax.ShapeDtypeStruct((N, DH), jnp.float32),
            jax.ShapeDtypeStruct((N, DH), jnp.float32),
        ),
        mesh=mesh,
        scratch_types=[
            pltpu.VMEM_SHARED((N, DH), jnp.float32),
            pltpu.VMEM((3, CH), jnp.int32),
            pltpu.VMEM((3, CH), jnp.int32),
            pltpu.VMEM((3, CH), jnp.int32),
            pltpu.VMEM((CH, DH), jnp.float32),
            pltpu.VMEM((CH, DH), jnp.float32),
            pltpu.VMEM((CH, DH), jnp.float32),
            pltpu.VMEM((CH, DH), jnp.float32),
            pltpu.SemaphoreType.DMA,
            pltpu.SemaphoreType.DMA,
            pltpu.SemaphoreType.DMA,
            pltpu.SemaphoreType.DMA,
            pltpu.SemaphoreType.DMA,
            pltpu.SemaphoreType.DMA,
        ],
    )
    def k(nf2_h, ef2_h, sidx0_h, sidx1_h, eidx0_h, eidx1_h, dst_h, zrows_h,
          out0_h, out1_h, acc, sib, eib, dib,
          nr0, er0, nr1, er1,
          isem0, isem1, isem2, gsem0, gsem1, ssem):
        c = lax.axis_index("c")
        s = lax.axis_index("s")
        nrbuf = (nr0, nr1)
        erbuf = (er0, er1)
        isem = (isem0, isem1, isem2)
        gsem = (gsem0, gsem1)

        def run(sidx_h, eidx_h, out_h):
            def idx_stage(kk, bi):
                base = s * EPT + kk * CH
                pltpu.async_copy(sidx_h.at[pl.ds(base, CH)], sib.at[bi],
                                 isem[bi])
                pltpu.async_copy(eidx_h.at[pl.ds(base, CH)], eib.at[bi],
                                 isem[bi])
                pltpu.async_copy(dst_h.at[pl.ds(base, CH)], dib.at[bi],
                                 isem[bi])

            def idx_wait(bi):
                pltpu.make_async_copy(sidx_h.at[pl.ds(0, CH)], sib.at[bi],
                                      isem[bi]).wait()
                pltpu.make_async_copy(eidx_h.at[pl.ds(0, CH)], eib.at[bi],
                                      isem[bi]).wait()
                pltpu.make_async_copy(dst_h.at[pl.ds(0, CH)], dib.at[bi],
                                      isem[bi]).wait()

            def gathers(bd, bi):
                pltpu.async_copy(nf2_h.at[sib.at[bi]], nrbuf[bd], gsem[bd])
                pltpu.async_copy(ef2_h.at[eib.at[bi]], erbuf[bd], gsem[bd])

            def gathers_wait(bd):
                pltpu.make_async_copy(nf2_h.at[pl.ds(0, CH)], nrbuf[bd],
                                      gsem[bd]).wait()
                pltpu.make_async_copy(ef2_h.at[pl.ds(0, CH)], erbuf[bd],
                                      gsem[bd]).wait()

            def scatter(bd, bi):
                a1 = pltpu.async_copy(nrbuf[bd], acc.at[dib.at[bi]], ssem,
                                      add=True)
                a2 = pltpu.async_copy(erbuf[bd], acc.at[dib.at[bi]], ssem,
                                      add=True)
                a1.wait()
                a2.wait()

            def sub(kk, bd, bi, do_next_g, do_next_i):
                if do_next_g:
                    idx_wait((bi + 1) % 3)
                    gathers((bd + 1) % 2, (bi + 1) % 3)
                if do_next_i:
                    idx_stage(kk + 2, (bi + 2) % 3)
                gathers_wait(bd)
                scatter(bd, bi)

            idx_stage(0, 0)
            idx_wait(0)
            gathers(0, 0)
            idx_stage(1, 1)

            @pl.when(s < 15)
            def _():
                pltpu.sync_copy(zrows_h.at[pl.ds(0, RPT)],
                                acc.at[pl.ds(s * RPT, RPT)])

            @pl.when(s == 15)
            def _():
                pltpu.sync_copy(zrows_h, acc.at[pl.ds(15 * RPT, RPT_LAST)])

            plsc.subcore_barrier()

            def six(i, carry):
                kk = 6 * i
                for j in range(6):
                    sub(kk + j, j % 2, j % 3, True, True)
                return carry

            nt = (NCHUNK - 4) // 6
            lax.fori_loop(0, nt, six, 0)
            for kk in range(6 * nt, NCHUNK):
                sub(kk, kk % 2, kk % 3, kk + 1 < NCHUNK, kk + 2 < NCHUNK)

            plsc.subcore_barrier()

            @pl.when(s < 15)
            def _():
                pltpu.sync_copy(acc.at[pl.ds(s * RPT, RPT)],
                                out_h.at[pl.ds(s * RPT, RPT)])

            @pl.when(s == 15)
            def _():
                pltpu.sync_copy(acc.at[pl.ds(15 * RPT, RPT_LAST)],
                                out_h.at[pl.ds(15 * RPT, RPT_LAST)])

        @pl.when(c == 0)
        def _():
            run(sidx0_h, eidx0_h, out0_h)

        @pl.when(c == 1)
        def _():
            run(sidx1_h, eidx1_h, out1_h)

    return k(nf2, ef2, sidx0, sidx1, eidx0, eidx1, dst, zrows)


RB = 2000


def _mlp_bn_kernel(a0_ref, a1_ref, w1_ref, b1_ref, w2_ref, b2_ref,
                   g_ref, bt_ref, o_ref, h_scr, st_scr):
    p = pl.program_id(0)
    i = pl.program_id(1)

    @pl.when(p == 0)
    def _():
        x = jnp.concatenate([a0_ref[...], a1_ref[...]],
                            axis=1).astype(jnp.bfloat16)
        h1 = jnp.maximum(
            jnp.dot(x, w1_ref[...], preferred_element_type=jnp.float32)
            + b1_ref[...], 0.0)
        h = (jnp.dot(h1.astype(jnp.bfloat16), w2_ref[...],
                     preferred_element_type=jnp.float32) + b2_ref[...])
        h_scr[pl.ds(i * RB, RB), :] = h
        s1 = jnp.sum(h, axis=0, keepdims=True)
        s2 = jnp.sum(h * h, axis=0, keepdims=True)

        @pl.when(i == 0)
        def _():
            st_scr[...] = jnp.zeros_like(st_scr)

        st_scr[0:2] += jnp.concatenate([s1, s2], axis=0)

    @pl.when(p == 1)
    def _():
        inv_n = 1.0 / N
        mean = st_scr[0:1] * inv_n
        var = st_scr[1:2] * inv_n - mean * mean
        rstd = lax.rsqrt(var + 1e-5)
        hb = h_scr[pl.ds(i * RB, RB), :]
        o_ref[...] = (hb - mean) * (rstd * g_ref[...]) + bt_ref[...]


def _mlp_bn(a0, a1, W1, b1, W2, b2, gamma, beta):
    grid = (2, N // RB)
    return pl.pallas_call(
        _mlp_bn_kernel,
        grid=grid,
        in_specs=[
            pl.BlockSpec((RB, DH), lambda p, i: (i, 0)),
            pl.BlockSpec((RB, DH), lambda p, i: (i, 0)),
            pl.BlockSpec((D, 2 * D), lambda p, i: (0, 0)),
            pl.BlockSpec((1, 2 * D), lambda p, i: (0, 0)),
            pl.BlockSpec((2 * D, D), lambda p, i: (0, 0)),
            pl.BlockSpec((1, D), lambda p, i: (0, 0)),
            pl.BlockSpec((1, D), lambda p, i: (0, 0)),
            pl.BlockSpec((1, D), lambda p, i: (0, 0)),
        ],
        out_specs=pl.BlockSpec((RB, D), lambda p, i: (i, 0)),
        out_shape=jax.ShapeDtypeStruct((N, D), jnp.float32),
        scratch_shapes=[
            pltpu.VMEM((N, D), jnp.float32),
            pltpu.VMEM((8, D), jnp.float32),
        ],
    )(a0, a1, W1.astype(jnp.bfloat16), b1.reshape(1, 2 * D),
      W2.astype(jnp.bfloat16), b2.reshape(1, D),
      gamma.reshape(1, D), beta.reshape(1, D))


def kernel(node_feats, edge_index, edge_feats, W1, b1, W2, b2, gamma, beta):
    src = edge_index[0]
    dst = edge_index[1]
    nf2 = node_feats.reshape(2 * N, DH)
    ef2 = edge_feats.reshape(2 * E, DH)
    sidx0 = src * 2
    sidx1 = sidx0 + 1
    eidx0 = jnp.arange(E, dtype=jnp.int32) * 2
    eidx1 = eidx0 + 1
    zrows = jnp.zeros((RPT_LAST, DH), jnp.float32)
    a0, a1 = _sc_segment_sum(nf2, ef2, sidx0, sidx1, eidx0, eidx1, dst, zrows)
    return _mlp_bn(a0, a1, W1, b1, W2, b2, gamma, beta)


# speedup vs baseline: 1.0248x; 1.0011x over previous
import functools

import jax
import jax.numpy as jnp
from jax import lax
from jax.experimental import pallas as pl
from jax.experimental.pallas import tpu as pltpu
from jax.experimental.pallas import tpu_sc as plsc

N = 10000
E = 160000
D = 256
DH = 128
NS = 16
CH = 80
EPT = E // NS
NCHUNK = EPT // CH
RPT = 624
RPT_LAST = N - 15 * RPT


def _sc_segment_sum(nf2, ef2, sidx0, sidx1, eidx0, eidx1, dst, zrows):
    mesh = plsc.VectorSubcoreMesh(core_axis_name="c", subcore_axis_name="s")

    @functools.partial(
        pl.kernel,
        out_type=(
            jax.ShapeDtypeStruct((N, DH), jnp.float32),
            jax.ShapeDtypeStruct((N, DH), jnp.float32),
        ),
        mesh=mesh,
        scratch_types=[
            pltpu.VMEM_SHARED((N, DH), jnp.float32),
            pltpu.VMEM((3, CH), jnp.int32),
            pltpu.VMEM((3, CH), jnp.int32),
            pltpu.VMEM((3, CH), jnp.int32),
            pltpu.VMEM((CH, DH), jnp.float32),
            pltpu.VMEM((CH, DH), jnp.float32),
            pltpu.VMEM((CH, DH), jnp.float32),
            pltpu.VMEM((CH, DH), jnp.float32),
            pltpu.SemaphoreType.DMA,
            pltpu.SemaphoreType.DMA,
            pltpu.SemaphoreType.DMA,
            pltpu.SemaphoreType.DMA,
            pltpu.SemaphoreType.DMA,
            pltpu.SemaphoreType.DMA,
        ],
    )
    def k(nf2_h, ef2_h, sidx0_h, sidx1_h, eidx0_h, eidx1_h, dst_h, zrows_h,
          out0_h, out1_h, acc, sib, eib, dib,
          nr0, er0, nr1, er1,
          isem0, isem1, isem2, gsem0, gsem1, ssem):
        c = lax.axis_index("c")
        s = lax.axis_index("s")
        nrbuf = (nr0, nr1)
        erbuf = (er0, er1)
        isem = (isem0, isem1, isem2)
        gsem = (gsem0, gsem1)

        def run(sidx_h, eidx_h, out_h):
            def idx_stage(kk, bi):
                base = s * EPT + kk * CH
                pltpu.async_copy(sidx_h.at[pl.ds(base, CH)], sib.at[bi],
                                 isem[bi])
                pltpu.async_copy(eidx_h.at[pl.ds(base, CH)], eib.at[bi],
                                 isem[bi])
                pltpu.async_copy(dst_h.at[pl.ds(base, CH)], dib.at[bi],
                                 isem[bi])

            def idx_wait(bi):
                pltpu.make_async_copy(sidx_h.at[pl.ds(0, CH)], sib.at[bi],
                                      isem[bi]).wait()
                pltpu.make_async_copy(eidx_h.at[pl.ds(0, CH)], eib.at[bi],
                                      isem[bi]).wait()
                pltpu.make_async_copy(dst_h.at[pl.ds(0, CH)], dib.at[bi],
                                      isem[bi]).wait()

            def gathers(bd, bi):
                pltpu.async_copy(nf2_h.at[sib.at[bi]], nrbuf[bd], gsem[bd])
                pltpu.async_copy(ef2_h.at[eib.at[bi]], erbuf[bd], gsem[bd])

            def gathers_wait(bd):
                pltpu.make_async_copy(nf2_h.at[pl.ds(0, CH)], nrbuf[bd],
                                      gsem[bd]).wait()
                pltpu.make_async_copy(ef2_h.at[pl.ds(0, CH)], erbuf[bd],
                                      gsem[bd]).wait()

            def scatter(bd, bi):
                a1 = pltpu.async_copy(nrbuf[bd], acc.at[dib.at[bi]], ssem,
                                      add=True)
                a2 = pltpu.async_copy(erbuf[bd], acc.at[dib.at[bi]], ssem,
                                      add=True)
                a1.wait()
                a2.wait()

            def sub(kk, bd, bi, do_next_g, do_next_i):
                if do_next_g:
                    idx_wait((bi + 1) % 3)
                    gathers((bd + 1) % 2, (bi + 1) % 3)
                if do_next_i:
                    idx_stage(kk + 2, (bi + 2) % 3)
                gathers_wait(bd)
                scatter(bd, bi)

            idx_stage(0, 0)
            idx_wait(0)
            gathers(0, 0)
            idx_stage(1, 1)

            @pl.when(s < 15)
            def _():
                pltpu.sync_copy(zrows_h.at[pl.ds(0, RPT)],
                                acc.at[pl.ds(s * RPT, RPT)])

            @pl.when(s == 15)
            def _():
                pltpu.sync_copy(zrows_h, acc.at[pl.ds(15 * RPT, RPT_LAST)])

            plsc.subcore_barrier()

            def six(i, carry):
                kk = 6 * i
                for j in range(6):
                    sub(kk + j, j % 2, j % 3, True, True)
                return carry

            nt = (NCHUNK - 4) // 6
            lax.fori_loop(0, nt, six, 0)
            for kk in range(6 * nt, NCHUNK):
                sub(kk, kk % 2, kk % 3, kk + 1 < NCHUNK, kk + 2 < NCHUNK)

            plsc.subcore_barrier()

            @pl.when(s < 15)
            def _():
                pltpu.sync_copy(acc.at[pl.ds(s * RPT, RPT)],
                                out_h.at[pl.ds(s * RPT, RPT)])

            @pl.when(s == 15)
            def _():
                pltpu.sync_copy(acc.at[pl.ds(15 * RPT, RPT_LAST)],
                                out_h.at[pl.ds(15 * RPT, RPT_LAST)])

        @pl.when(c == 0)
        def _():
            run(sidx0_h, eidx0_h, out0_h)

        @pl.when(c == 1)
        def _():
            run(sidx1_h, eidx1_h, out1_h)

    return k(nf2, ef2, sidx0, sidx1, eidx0, eidx1, dst, zrows)


RB = 2000


def _mlp_bn_kernel(a0_ref, a1_ref, w1_ref, b1_ref, w2_ref, b2_ref,
                   g_ref, bt_ref, o_ref, h_scr, st_scr):
    p = pl.program_id(0)
    i = pl.program_id(1)

    @pl.when(p == 0)
    def _():
        x = jnp.concatenate([a0_ref[...], a1_ref[...]],
                            axis=1).astype(jnp.bfloat16)
        h1 = jnp.maximum(
            jnp.dot(x, w1_ref[...], preferred_element_type=jnp.float32)
            + b1_ref[...], 0.0)
        h = (jnp.dot(h1.astype(jnp.bfloat16), w2_ref[...],
                     preferred_element_type=jnp.float32) + b2_ref[...])
        h_scr[pl.ds(i * RB, RB), :] = h
        s1 = jnp.sum(h, axis=0, keepdims=True)
        s2 = jnp.sum(h * h, axis=0, keepdims=True)

        @pl.when(i == 0)
        def _():
            st_scr[...] = jnp.zeros_like(st_scr)

        st_scr[0:2] += jnp.concatenate([s1, s2], axis=0)

    @pl.when(p == 1)
    def _():
        inv_n = 1.0 / N
        mean = st_scr[0:1] * inv_n
        var = st_scr[1:2] * inv_n - mean * mean
        rstd = lax.rsqrt(var + 1e-5)
        hb = h_scr[pl.ds(i * RB, RB), :]
        o_ref[...] = (hb - mean) * (rstd * g_ref[...]) + bt_ref[...]


def _mlp_bn(a0, a1, W1, b1, W2, b2, gamma, beta):
    grid = (2, N // RB)
    return pl.pallas_call(
        _mlp_bn_kernel,
        grid=grid,
        in_specs=[
            pl.BlockSpec((RB, DH), lambda p, i: (i, 0)),
            pl.BlockSpec((RB, DH), lambda p, i: (i, 0)),
            pl.BlockSpec((D, 2 * D), lambda p, i: (0, 0)),
            pl.BlockSpec((1, 2 * D), lambda p, i: (0, 0)),
            pl.BlockSpec((2 * D, D), lambda p, i: (0, 0)),
            pl.BlockSpec((1, D), lambda p, i: (0, 0)),
            pl.BlockSpec((1, D), lambda p, i: (0, 0)),
            pl.BlockSpec((1, D), lambda p, i: (0, 0)),
        ],
        out_specs=pl.BlockSpec((RB, D), lambda p, i: (i, 0)),
        out_shape=jax.ShapeDtypeStruct((N, D), jnp.float32),
        scratch_shapes=[
            pltpu.VMEM((N, D), jnp.float32),
            pltpu.VMEM((8, D), jnp.float32),
        ],
    )(a0, a1, W1.astype(jnp.bfloat16), b1.reshape(1, 2 * D),
      W2.astype(jnp.bfloat16), b2.reshape(1, D),
      gamma.reshape(1, D), beta.reshape(1, D))


def kernel(node_feats, edge_index, edge_feats, W1, b1, W2, b2, gamma, beta):
    src = edge_index[0]
    dst = edge_index[1]
    nf2 = node_feats.reshape(2 * N, DH)
    ef2 = edge_feats.reshape(2 * E, DH)
    sidx0 = src * 2
    sidx1 = sidx0 + 1
    eidx0 = jnp.arange(E, dtype=jnp.int32) * 2
    eidx1 = eidx0 + 1
    zrows = jnp.zeros((RPT_LAST, DH), jnp.float32)
    a0, a1 = _sc_segment_sum(nf2, ef2, sidx0, sidx1, eidx0, eidx1, dst, zrows)
    return _mlp_bn(a0, a1, W1, b1, W2, b2, gamma, beta)
